# baseline (speedup 1.0000x reference)
import functools

import jax
import jax.numpy as jnp
import numpy as np
from jax import lax
from jax.experimental import pallas as pl
from jax.experimental.pallas import tpu as pltpu
from jax.experimental.pallas import tpu_sc as plsc

NUM_FIELDS = 26
DEPTH = 1000
BATCH = 4096
OUT_COLS = NUM_FIELDS * DEPTH

NUM_WORKERS = 32
ROWS_PER_WORKER = BATCH // NUM_WORKERS
ROWS_PER_CHUNK = 4
CHUNKS = ROWS_PER_WORKER // ROWS_PER_CHUNK
IDX_PER_WORKER = ROWS_PER_WORKER * NUM_FIELDS
IDX_PER_CHUNK = ROWS_PER_CHUNK * NUM_FIELDS
VECS_PER_CHUNK = (IDX_PER_CHUNK + 15) // 16
CHUNK_WORDS = ROWS_PER_CHUNK * OUT_COLS
TRASH = CHUNK_WORDS
BUF_WORDS = 105088

_jj = np.arange(VECS_PER_CHUNK * 16)
_base = np.where(
    _jj < IDX_PER_CHUNK,
    (_jj // NUM_FIELDS) * OUT_COLS + (_jj % NUM_FIELDS) * DEPTH,
    TRASH,
).astype(np.int32)
_pad_vals = np.arange(16, dtype=np.int32)

_mesh = plsc.VectorSubcoreMesh(core_axis_name="c", subcore_axis_name="s")


@functools.partial(
    pl.kernel,
    mesh=_mesh,
    out_type=jax.ShapeDtypeStruct((BATCH * OUT_COLS,), jnp.float32),
    scratch_types=[
        pltpu.VMEM((IDX_PER_WORKER + 16,), jnp.int32),
        pltpu.VMEM((VECS_PER_CHUNK * 16,), jnp.int32),
        pltpu.VMEM((BUF_WORDS,), jnp.float32),
    ],
    compiler_params=pltpu.CompilerParams(needs_layout_passes=False),
)
def _onehot_sc(fv_hbm, base_hbm, out_hbm, idx_v, base_v, buf_v):
    wid = lax.axis_index("s") * 2 + lax.axis_index("c")

    pltpu.sync_copy(
        fv_hbm.at[pl.ds(wid * IDX_PER_WORKER, IDX_PER_WORKER)],
        idx_v.at[pl.ds(0, IDX_PER_WORKER)],
    )
    pltpu.sync_copy(
        fv_hbm.at[pl.ds(NUM_WORKERS * IDX_PER_WORKER, 16)],
        idx_v.at[pl.ds(IDX_PER_WORKER, 16)],
    )
    pltpu.sync_copy(base_hbm, base_v)

    t16 = idx_v[pl.ds(0, 16)]
    zeros16 = (t16 * 0).astype(jnp.float32)
    ones16 = zeros16 + 1.0

    def _zero(i, _):
        for u in range(8):
            buf_v[pl.ds(i * 128 + u * 16, 16)] = zeros16
        return _

    lax.fori_loop(0, BUF_WORDS // 128, _zero, None)

    def _chunk(c, _):
        jbase = c * IDX_PER_CHUNK
        positions = []
        for k in range(VECS_PER_CHUNK):
            v = idx_v[pl.ds(jbase + k * 16, 16)]
            pos = base_v[pl.ds(k * 16, 16)] + v
            positions.append(pos)
            plsc.store_scatter(buf_v, [pos], ones16)
        pltpu.sync_copy(
            buf_v.at[pl.ds(0, CHUNK_WORDS)],
            out_hbm.at[pl.ds((wid * ROWS_PER_WORKER + c * ROWS_PER_CHUNK) * OUT_COLS,
                             CHUNK_WORDS)],
        )
        for pos in positions:
            plsc.store_scatter(buf_v, [pos], zeros16)
        return _

    lax.fori_loop(0, CHUNKS, _chunk, None)


def kernel(feature_value):
    fv_flat = jnp.concatenate(
        [feature_value.reshape(-1), jnp.asarray(_pad_vals)])
    out = _onehot_sc(fv_flat, jnp.asarray(_base))
    return out.reshape(BATCH, OUT_COLS)
